# baseline (speedup 1.0000x reference)
import jax
import jax.numpy as jnp
from jax import lax
from jax.experimental import pallas as pl
from jax.experimental.pallas import tpu as pltpu
from jax.experimental.pallas import tpu_sc as plsc

_N_MEM = 16
_LANES = 16
_B_PER_W = 32
_N_INPUTS = 128
_N_RULES = 512
_POS = _N_INPUTS * _N_MEM


def _sc_body(fx_hbm, sel_hbm, out_hbm, chunk_v, sel_v, out_v):
    wid = lax.axis_index("s") * 2 + lax.axis_index("c")
    b0 = wid * _B_PER_W
    pltpu.sync_copy(fx_hbm.at[pl.ds(b0 * _POS, _B_PER_W * _POS)], chunk_v)
    half = jnp.full((_LANES,), 0.5, jnp.float32)
    one = jnp.full((_LANES,), 1.0, jnp.float32)

    def quarter_body(q, carry):
        pltpu.sync_copy(sel_hbm.at[:, pl.ds(q * 128, 128)], sel_v)

        def rgroup_body(rgl, carry2):
            def i_body(i, accs):
                y = sel_v[i, pl.ds(rgl * _LANES, _LANES)] * jnp.float32(_N_MEM)
                f = y.astype(jnp.int32)
                frac = y - f.astype(jnp.float32)
                m = (f + jnp.where(frac > half, 1, 0)
                     + jnp.where(frac == half, f & 1, 0))
                used = m < _N_MEM
                p = jnp.minimum(m, _N_MEM - 1) + i * _N_MEM
                new = []
                for b in range(_B_PER_W):
                    g = plsc.load_gather(chunk_v, [p + b * _POS], mask=used)
                    new.append(accs[b] * jnp.where(used, g, one))
                return tuple(new)

            init = tuple(jnp.full((_LANES,), 1.0, jnp.float32)
                         for _ in range(_B_PER_W))
            accs = lax.fori_loop(0, _N_INPUTS, i_body, init)
            rg = q * 8 + rgl
            for b in range(_B_PER_W):
                out_v[b, pl.ds(rg * _LANES, _LANES)] = accs[b]
            return carry2

        lax.fori_loop(0, 8, rgroup_body, 0)
        return carry

    lax.fori_loop(0, _N_RULES // 128, quarter_body, 0)
    pltpu.sync_copy(out_v, out_hbm.at[pl.ds(b0, _B_PER_W), :])


def kernel(fuzzified_x, input_selectors):
    b = fuzzified_x.shape[0]
    fx_flat = fuzzified_x.reshape(b * _POS)
    mesh = plsc.VectorSubcoreMesh(core_axis_name="c", subcore_axis_name="s")
    f = pl.kernel(
        _sc_body,
        out_type=jax.ShapeDtypeStruct((b, _N_RULES), jnp.float32),
        mesh=mesh,
        compiler_params=pltpu.CompilerParams(needs_layout_passes=False),
        scratch_types=[
            pltpu.VMEM((_B_PER_W * _POS,), jnp.float32),
            pltpu.VMEM((_N_INPUTS, 128), jnp.float32),
            pltpu.VMEM((_B_PER_W, _N_RULES), jnp.float32),
        ],
    )
    return f(fx_flat, input_selectors)
